# Initial kernel scaffold; baseline (speedup 1.0000x reference)
import jax
import jax.numpy as jnp
from jax.experimental import pallas as pl


def kernel(x, edge_index, W, b):
    raise NotImplementedError("write your pallas kernel here")


# speedup vs baseline: 13.3886x; 13.3886x over previous
import functools

import jax
import jax.numpy as jnp
from jax import lax
from jax.experimental import pallas as pl
from jax.experimental.pallas import tpu as pltpu
from jax.experimental.pallas import tpu_sc as plsc

N = 10000
E = 320000
D = 128
NCORES = 2
NSUB = 16
NTILES = NCORES * NSUB
CHUNK = 128
NCHUNK = 79
EDGES_PER_TILE = CHUNK * NCHUNK
E_PAD = NTILES * EDGES_PER_TILE
NSH = 10240
BR = 1000

_ZC5 = ((0, 128), (1, 128), (2, 128), (3, 128), (4, 128))


def _mesh():
    return plsc.VectorSubcoreMesh(core_axis_name="c", subcore_axis_name="s")


@functools.partial(
    pl.kernel,
    out_type=jax.ShapeDtypeStruct((NCORES, NSH), jnp.float32),
    scratch_types=[
        pltpu.VMEM((CHUNK,), jnp.float32),
        pltpu.VMEM((CHUNK,), jnp.int32),
        pltpu.VMEM((640,), jnp.float32),
        pltpu.VMEM_SHARED((NSH,), jnp.float32),
    ],
    mesh=_mesh(),
)
def _deg_call(rp_hbm, deg_hbm, ones_v, ridx_v, zbuf, deg_sh):
    c = lax.axis_index("c")
    s = lax.axis_index("s")
    wid = c * NSUB + s
    for j in range(CHUNK // 16):
        ones_v[pl.ds(j * 16, 16)] = jnp.ones((16,), jnp.float32)
    for j in range(640 // 16):
        zbuf[pl.ds(j * 16, 16)] = jnp.zeros((16,), jnp.float32)
    pltpu.sync_copy(zbuf, deg_sh.at[pl.ds(s * 640, 640)])
    plsc.subcore_barrier()

    base = wid * EDGES_PER_TILE

    def body(k, carry):
        pltpu.sync_copy(rp_hbm.at[pl.ds(base + k * CHUNK, CHUNK)], ridx_v)
        pltpu.sync_copy(ones_v, deg_sh.at[ridx_v], add=True)
        return carry

    lax.fori_loop(0, NCHUNK, body, 0)
    plsc.subcore_barrier()
    pltpu.sync_copy(deg_sh.at[pl.ds(s * 640, 640)],
                    deg_hbm.at[c, pl.ds(s * 640, 640)])


@functools.partial(
    pl.kernel,
    out_type=jax.ShapeDtypeStruct((NCORES, NSH, D), jnp.float32),
    scratch_types=[
        pltpu.VMEM((CHUNK,), jnp.int32),
        pltpu.VMEM((CHUNK,), jnp.int32),
        pltpu.VMEM((CHUNK, D), jnp.float32),
        pltpu.VMEM_SHARED((NSH, D), jnp.float32),
        pltpu.SemaphoreType.DMA,
    ],
    mesh=_mesh(),
)
def _msg_call(cp_hbm, rp_hbm, h_hbm, z_hbm, out_hbm,
              col_v, row_v, msg_v, acc_sh, sem):
    c = lax.axis_index("c")
    s = lax.axis_index("s")
    wid = c * NSUB + s
    for k, sz in _ZC5:
        pltpu.sync_copy(z_hbm.at[pl.ds(0, sz)],
                        acc_sh.at[pl.ds(s * 640 + k * 128, sz)])
    plsc.subcore_barrier()

    base = wid * EDGES_PER_TILE

    def body(k, carry):
        off = base + k * CHUNK
        pltpu.sync_copy(cp_hbm.at[pl.ds(off, CHUNK)], col_v)
        pltpu.sync_copy(rp_hbm.at[pl.ds(off, CHUNK)], row_v)
        pltpu.async_copy(h_hbm.at[col_v], msg_v, sem).wait()
        pltpu.sync_copy(msg_v, acc_sh.at[row_v], add=True)
        return carry

    lax.fori_loop(0, NCHUNK, body, 0)
    plsc.subcore_barrier()
    for k, sz in _ZC5:
        r0 = s * 640 + k * 128
        pltpu.sync_copy(acc_sh.at[pl.ds(r0, sz)], out_hbm.at[c, pl.ds(r0, sz)])


def _mm_body(x_ref, w_ref, d0_ref, d1_ref, h_ref):
    deg = d0_ref[...] + d1_ref[...]
    dinv = jnp.where(deg > 0, lax.rsqrt(jnp.maximum(deg, 1e-12)), 0.0)
    h_ref[...] = jnp.dot(x_ref[...], w_ref[...],
                         preferred_element_type=jnp.float32) * dinv


def _fin_body(p0_ref, p1_ref, d0_ref, d1_ref, b_ref, o_ref):
    deg = d0_ref[...] + d1_ref[...]
    dinv = jnp.where(deg > 0, lax.rsqrt(jnp.maximum(deg, 1e-12)), 0.0)
    o_ref[...] = jnp.maximum((p0_ref[...] + p1_ref[...]) * dinv + b_ref[...],
                             0.0)


def kernel(x, edge_index, W, b):
    row = edge_index[0]
    col = edge_index[1]
    pad = E_PAD - E
    rp = jnp.concatenate([row, jnp.full((pad,), N, jnp.int32)])
    cp = jnp.concatenate([col, jnp.zeros((pad,), jnp.int32)])
    z128 = jnp.zeros((CHUNK, D), jnp.float32)

    deg2 = _deg_call(rp)
    d0 = deg2[0, :N].reshape(N, 1)
    d1 = deg2[1, :N].reshape(N, 1)

    h = pl.pallas_call(
        _mm_body,
        grid=(N // BR,),
        in_specs=[
            pl.BlockSpec((BR, D), lambda i: (i, 0)),
            pl.BlockSpec((D, D), lambda i: (0, 0)),
            pl.BlockSpec((BR, 1), lambda i: (i, 0)),
            pl.BlockSpec((BR, 1), lambda i: (i, 0)),
        ],
        out_specs=pl.BlockSpec((BR, D), lambda i: (i, 0)),
        out_shape=jax.ShapeDtypeStruct((N, D), jnp.float32),
    )(x, W, d0, d1)

    parts = _msg_call(cp, rp, h, z128)

    out = pl.pallas_call(
        _fin_body,
        grid=(N // BR,),
        in_specs=[
            pl.BlockSpec((BR, D), lambda i: (i, 0)),
            pl.BlockSpec((BR, D), lambda i: (i, 0)),
            pl.BlockSpec((BR, 1), lambda i: (i, 0)),
            pl.BlockSpec((BR, 1), lambda i: (i, 0)),
            pl.BlockSpec((1, D), lambda i: (0, 0)),
        ],
        out_specs=pl.BlockSpec((BR, D), lambda i: (i, 0)),
        out_shape=jax.ShapeDtypeStruct((N, D), jnp.float32),
    )(parts[0, :N], parts[1, :N], d0, d1, b.reshape(1, D))
    return out
